# baseline (speedup 1.0000x reference)
import functools

import jax
import jax.numpy as jnp
from jax import lax
from jax.experimental import pallas as pl
from jax.experimental.pallas import tpu as pltpu
from jax.experimental.pallas import tpu_sc as plsc

_NBUF = 4


def _gather_sc(x5, weight, nb, t, d):
    info = plsc.get_sparse_core_info()
    nc, ns = info.num_cores, info.num_subcores
    nw = nc * ns
    assert nb // 128 == nw
    ntr = t // 8

    mesh = plsc.VectorSubcoreMesh(core_axis_name="c", subcore_axis_name="s")

    @functools.partial(
        pl.kernel,
        out_type=jax.ShapeDtypeStruct((t, d // 8, nw, 8, 128), jnp.float32),
        mesh=mesh,
        scratch_types=[
            pltpu.VMEM((ntr, 8, 128), jnp.int32),
            pltpu.VMEM((_NBUF, 128, d), jnp.float32),
            pltpu.VMEM((_NBUF, d // 8, 8, 128), jnp.float32),
            pltpu.SemaphoreType.DMA((_NBUF,)),
            pltpu.SemaphoreType.DMA((_NBUF,)),
        ],
        compiler_params=pltpu.CompilerParams(needs_layout_passes=False, use_tc_tiling_on_sc=False),
    )
    def k(x5_hbm, table_hbm, out5_hbm, idx_v, rows_v, tbuf_v, gsem, osem):
        wid = lax.axis_index("s") * nc + lax.axis_index("c")
        pltpu.sync_copy(x5_hbm.at[:, wid], idx_v)


        def fire_gather(tt, u):
            tr = lax.div(tt, 8)
            ti = lax.rem(tt, 8)
            pltpu.async_copy(
                table_hbm.at[idx_v.at[tr, ti]], rows_v.at[u], gsem.at[u]
            )

        def wait_gather(u):
            pltpu.make_async_copy(
                table_hbm.at[idx_v.at[0, 0]], rows_v.at[u], gsem.at[u]
            ).wait()

        def fire_write(tt, u):
            pltpu.async_copy(tbuf_v.at[u], out5_hbm.at[tt, :, wid], osem.at[u])

        def wait_write(u):
            pltpu.make_async_copy(
                tbuf_v.at[u], out5_hbm.at[0, :, wid], osem.at[u]
            ).wait()

        def transpose(u):
            base = lax.iota(jnp.int32, 16)
            for c in range(d):
                col_id = jnp.full((16,), c, jnp.int32)
                for kk in range(8):
                    v = plsc.load_gather(
                        rows_v.at[u], [base + 16 * kk, col_id]
                    )
                    tbuf_v[u, c // 8, c % 8, pl.ds(kk * 16, 16)] = v

        n_groups = t // _NBUF

        for u in range(_NBUF):
            fire_gather(jnp.int32(u), u)
        for u in range(_NBUF):
            wait_gather(u)
            transpose(u)
            fire_write(jnp.int32(u), u)
            fire_gather(jnp.int32(_NBUF + u), u)

        def body(g, carry):
            t0 = g * _NBUF
            for u in range(_NBUF):
                tt = t0 + u
                wait_write(u)
                wait_gather(u)
                transpose(u)
                fire_write(tt, u)
                fire_gather(tt + _NBUF, u)
            return carry

        lax.fori_loop(1, n_groups - 1, body, 0)

        t0 = (n_groups - 1) * _NBUF
        for u in range(_NBUF):
            wait_write(u)
            wait_gather(u)
            transpose(u)
            fire_write(jnp.int32(t0 + u), u)
        for u in range(_NBUF):
            wait_write(u)

    return k(x5, weight)


def kernel(x, weight):
    nb, t = x.shape
    v, d = weight.shape
    x5 = x.reshape(nb // 128, 128, t // 8, 8).transpose(2, 0, 3, 1)
    out5 = _gather_sc(x5, weight, nb, t, d)
    return out5.transpose(2, 4, 0, 1, 3).reshape(nb, t, d)


# speedup vs baseline: 1.2373x; 1.2373x over previous
import functools

import jax
import jax.numpy as jnp
from jax import lax
from jax.experimental import pallas as pl
from jax.experimental.pallas import tpu as pltpu
from jax.experimental.pallas import tpu_sc as plsc

_NBUF = 4


def _gather_sc(x5, weight, nb, t, d):
    info = plsc.get_sparse_core_info()
    nc, ns = info.num_cores, info.num_subcores
    nw = nc * ns
    assert nb // 128 == nw
    ntr = t // 8

    mesh = plsc.VectorSubcoreMesh(core_axis_name="c", subcore_axis_name="s")

    @functools.partial(
        pl.kernel,
        out_type=jax.ShapeDtypeStruct((t, d // 8, nw, 8, 128), jnp.float32),
        mesh=mesh,
        scratch_types=[
            pltpu.VMEM((ntr, 8, 128), jnp.int32),
            pltpu.VMEM((_NBUF, 128, d), jnp.float32),
            pltpu.VMEM((_NBUF, d // 8, 8, 128), jnp.float32),
            pltpu.SemaphoreType.DMA((_NBUF,)),
            pltpu.SemaphoreType.DMA((_NBUF,)),
        ],
        compiler_params=pltpu.CompilerParams(needs_layout_passes=False, use_tc_tiling_on_sc=False),
    )
    def k(x5_hbm, table_hbm, out5_hbm, idx_v, rows_v, tbuf_v, gsem, osem):
        wid = lax.axis_index("s") * nc + lax.axis_index("c")
        pltpu.sync_copy(x5_hbm.at[:, wid], idx_v)


        def fire_gather(tt, u):
            tr = lax.div(tt, 8)
            ti = lax.rem(tt, 8)
            pltpu.async_copy(
                table_hbm.at[idx_v.at[tr, ti]], rows_v.at[u], gsem.at[u]
            )

        def wait_gather(u):
            pltpu.make_async_copy(
                table_hbm.at[idx_v.at[0, 0]], rows_v.at[u], gsem.at[u]
            ).wait()

        def fire_write(tt, u):
            pltpu.async_copy(tbuf_v.at[u], out5_hbm.at[tt, :, wid], osem.at[u])

        def wait_write(u):
            pltpu.make_async_copy(
                tbuf_v.at[u], out5_hbm.at[0, :, wid], osem.at[u]
            ).wait()

        def transpose(u):
            base = lax.iota(jnp.int32, 16)
            for c0 in range(0, d, 2):
                vs = []
                for c in (c0, c0 + 1):
                    col_id = jnp.full((16,), c, jnp.int32)
                    for kk in range(8):
                        vs.append(
                            (c, kk,
                             plsc.load_gather(
                                 rows_v.at[u], [base + 16 * kk, col_id]
                             ))
                        )
                for c, kk, v in vs:
                    tbuf_v[u, c // 8, c % 8, pl.ds(kk * 16, 16)] = v

        n_groups = t // _NBUF

        for u in range(_NBUF):
            fire_gather(jnp.int32(u), u)
        for u in range(_NBUF):
            wait_gather(u)
            transpose(u)
            fire_write(jnp.int32(u), u)
            fire_gather(jnp.int32(_NBUF + u), u)

        def body(g, carry):
            t0 = g * _NBUF
            for u in range(_NBUF):
                tt = t0 + u
                wait_write(u)
                wait_gather(u)
                transpose(u)
                fire_write(tt, u)
                fire_gather(tt + _NBUF, u)
            return carry

        lax.fori_loop(1, n_groups - 1, body, 0)

        t0 = (n_groups - 1) * _NBUF
        for u in range(_NBUF):
            wait_write(u)
            wait_gather(u)
            transpose(u)
            fire_write(jnp.int32(t0 + u), u)
        for u in range(_NBUF):
            wait_write(u)

    return k(x5, weight)


def kernel(x, weight):
    nb, t = x.shape
    v, d = weight.shape
    x5 = x.reshape(nb // 128, 128, t // 8, 8).transpose(2, 0, 3, 1)
    out5 = _gather_sc(x5, weight, nb, t, d)
    return out5.transpose(2, 4, 0, 1, 3).reshape(nb, t, d)


# speedup vs baseline: 1.2788x; 1.0336x over previous
import functools

import jax
import jax.numpy as jnp
from jax import lax
from jax.experimental import pallas as pl
from jax.experimental.pallas import tpu as pltpu
from jax.experimental.pallas import tpu_sc as plsc

_GT = 8
_NG = 200 // _GT
_GI = _GT * 128


def _gather_sc(x6, weight, d):
    info = plsc.get_sparse_core_info()
    nc, ns = info.num_cores, info.num_subcores
    nw = nc * ns

    mesh = plsc.VectorSubcoreMesh(core_axis_name="c", subcore_axis_name="s")

    @functools.partial(
        pl.kernel,
        out_type=jax.ShapeDtypeStruct((200, d // 8, nw, 8, 128), jnp.float32),
        mesh=mesh,
        scratch_types=[
            pltpu.VMEM((2, _GI), jnp.int32),
            pltpu.VMEM((2 * _GI, d), jnp.float32),
            pltpu.VMEM((2, _GT // 2, d // 8, 8, 128), jnp.float32),
            pltpu.SemaphoreType.DMA((2,)),
            pltpu.SemaphoreType.DMA((2,)),
            pltpu.SemaphoreType.DMA((2,)),
        ],
        compiler_params=pltpu.CompilerParams(
            needs_layout_passes=False, use_tc_tiling_on_sc=False
        ),
    )
    def k(x6_hbm, table_hbm, out5_hbm, idx_v, rows_v, tbuf_v, isem, gsem, wsem):
        wid = lax.axis_index("s") * nc + lax.axis_index("c")

        def fire_idx(g, u):
            pltpu.async_copy(x6_hbm.at[g, wid], idx_v.at[u], isem.at[u])

        def wait_idx(u):
            pltpu.make_async_copy(
                x6_hbm.at[0, wid], idx_v.at[u], isem.at[u]
            ).wait()

        def fire_gather(u):
            pltpu.async_copy(
                table_hbm.at[idx_v.at[u]],
                rows_v.at[pl.ds(u * _GI, _GI)],
                gsem.at[u],
            )

        def wait_gather(u):
            pltpu.make_async_copy(
                table_hbm.at[idx_v.at[0]],
                rows_v.at[pl.ds(0, _GI)],
                gsem.at[u],
            ).wait()

        def fire_write(g, half):
            pltpu.async_copy(
                tbuf_v.at[half],
                out5_hbm.at[
                    pl.ds(g * _GT + half * (_GT // 2), _GT // 2), :, wid
                ],
                wsem.at[half],
            )

        def wait_write(half):
            pltpu.make_async_copy(
                tbuf_v.at[half],
                out5_hbm.at[pl.ds(0, _GT // 2), :, wid],
                wsem.at[half],
            ).wait()

        def transpose(u, half):
            base = lax.iota(jnp.int32, 16)
            off0 = u * _GI + half * (_GT // 2) * 128
            for jl in range(_GT // 2):
                for c0 in range(0, d, 4):
                    vs = []
                    for c in range(c0, c0 + 4):
                        col = jnp.full((16,), c, jnp.int32)
                        for kk in range(8):
                            bvec = base + (off0 + jl * 128 + kk * 16)
                            vs.append(
                                (c, kk, plsc.load_gather(rows_v, [bvec, col]))
                            )
                    for c, kk, v in vs:
                        tbuf_v[half, jl, c // 8, c % 8, pl.ds(kk * 16, 16)] = v

        fire_idx(jnp.int32(0), 0)
        fire_idx(jnp.int32(1), 1)
        wait_idx(0)
        fire_gather(0)
        wait_idx(1)
        fire_gather(1)
        fire_write(jnp.int32(0), 0)
        fire_write(jnp.int32(0), 1)

        def body(g, carry):
            u = lax.rem(g, 2)
            wait_gather(u)

            @pl.when(g <= _NG - 3)
            def _():
                fire_idx(g + 2, u)

            for half in (0, 1):
                wait_write(half)
                transpose(u, half)
                fire_write(g, half)

            @pl.when(g <= _NG - 3)
            def _():
                wait_idx(u)
                fire_gather(u)

            return carry

        lax.fori_loop(0, _NG, body, 0)
        wait_write(0)
        wait_write(1)

    return k(x6, weight)


def kernel(x, weight):
    nb, t = x.shape
    v, d = weight.shape
    x6 = (
        x.reshape(nb // 128, 128, t // 8, 8)
        .transpose(2, 0, 3, 1)
        .reshape(t // 8, nb // 128, 8 * 128)
    )
    out5 = _gather_sc(x6, weight, d)
    return out5.transpose(2, 4, 0, 1, 3).reshape(nb, t, d)


# speedup vs baseline: 1.5711x; 1.2285x over previous
import functools

import jax
import jax.numpy as jnp
from jax import lax
from jax.experimental import pallas as pl
from jax.experimental.pallas import tpu as pltpu
from jax.experimental.pallas import tpu_sc as plsc

_GT = 8
_NG = 200 // _GT
_GI = _GT * 128


def _gather_sc(x6, weight, d):
    info = plsc.get_sparse_core_info()
    nc, ns = info.num_cores, info.num_subcores
    nw = nc * ns

    mesh = plsc.VectorSubcoreMesh(core_axis_name="c", subcore_axis_name="s")

    @functools.partial(
        pl.kernel,
        out_type=jax.ShapeDtypeStruct((200, d // 8, nw, 8, 128), jnp.float32),
        mesh=mesh,
        scratch_types=[
            pltpu.VMEM((2, _GI), jnp.int32),
            pltpu.VMEM((2 * _GI, d), jnp.float32),
            pltpu.VMEM((2, _GT // 2, d // 8, 8, 128), jnp.float32),
            pltpu.SemaphoreType.DMA((2,)),
            pltpu.SemaphoreType.DMA((2,)),
            pltpu.SemaphoreType.DMA((2,)),
        ],
        compiler_params=pltpu.CompilerParams(
            needs_layout_passes=False, use_tc_tiling_on_sc=False
        ),
    )
    def k(x6_hbm, table_hbm, out5_hbm, idx_v, rows_v, tbuf_v, isem, gsem, wsem):
        wid = lax.axis_index("s") * nc + lax.axis_index("c")

        def fire_idx(g, u):
            pltpu.async_copy(x6_hbm.at[g, wid], idx_v.at[u], isem.at[u])

        def wait_idx(u):
            pltpu.make_async_copy(
                x6_hbm.at[0, wid], idx_v.at[u], isem.at[u]
            ).wait()

        def fire_gather(u):
            pltpu.async_copy(
                table_hbm.at[idx_v.at[u]],
                rows_v.at[pl.ds(u * _GI, _GI)],
                gsem.at[u],
            )

        def wait_gather(u):
            pltpu.make_async_copy(
                table_hbm.at[idx_v.at[0]],
                rows_v.at[pl.ds(0, _GI)],
                gsem.at[u],
            ).wait()

        def fire_write(g, half):
            pltpu.async_copy(
                tbuf_v.at[half],
                out5_hbm.at[
                    pl.ds(g * _GT + half * (_GT // 2), _GT // 2), :, wid
                ],
                wsem.at[half],
            )

        def wait_write(half):
            pltpu.make_async_copy(
                tbuf_v.at[half],
                out5_hbm.at[pl.ds(0, _GT // 2), :, wid],
                wsem.at[half],
            ).wait()

        def transpose(u, half):
            iot = lax.iota(jnp.int32, 16)
            off0 = u * _GI + half * (_GT // 2) * 128
            for jl in range(_GT // 2):
                for c0 in range(0, d, 4):
                    vs = []
                    for c in range(c0, c0 + 4):
                        colv = lax.bitwise_and(iot + c, d - 1)
                        crv = lax.shift_right_logical(colv, 3)
                        civ = lax.bitwise_and(colv, 7)
                        for b0 in range(0, 128, 16):
                            bvec = iot + (off0 + jl * 128 + b0)
                            v = plsc.load_gather(rows_v, [bvec, colv])
                            vs.append((crv, civ, b0, v))
                    for crv, civ, b0, v in vs:
                        plsc.store_scatter(
                            tbuf_v.at[half, jl], [crv, civ, iot + b0], v
                        )

        fire_idx(jnp.int32(0), 0)
        fire_idx(jnp.int32(1), 1)
        wait_idx(0)
        fire_gather(0)
        wait_idx(1)
        fire_gather(1)
        fire_write(jnp.int32(0), 0)
        fire_write(jnp.int32(0), 1)

        def body(g, carry):
            u = lax.rem(g, 2)
            wait_gather(u)

            @pl.when(g <= _NG - 3)
            def _():
                fire_idx(g + 2, u)

            for half in (0, 1):
                wait_write(half)
                transpose(u, half)
                fire_write(g, half)

            @pl.when(g <= _NG - 3)
            def _():
                wait_idx(u)
                fire_gather(u)

            return carry

        lax.fori_loop(0, _NG, body, 0)
        wait_write(0)
        wait_write(1)

    return k(x6, weight)


def kernel(x, weight):
    nb, t = x.shape
    v, d = weight.shape
    x6 = (
        x.reshape(nb // 128, 128, t // 8, 8)
        .transpose(2, 0, 3, 1)
        .reshape(t // 8, nb // 128, 8 * 128)
    )
    out5 = _gather_sc(x6, weight, d)
    return out5.transpose(2, 4, 0, 1, 3).reshape(nb, t, d)
